# Initial kernel scaffold; baseline (speedup 1.0000x reference)
import jax
import jax.numpy as jnp
from jax.experimental import pallas as pl


def kernel(nodes, edge_index, etypes, node_feat, bases0, comp0, wself0, bias0, gamma0, beta0, bases1, comp1, wself1, bias1, gamma1, beta1):
    raise NotImplementedError("write your pallas kernel here")


# speedup vs baseline: 2.6012x; 2.6012x over previous
import functools

import jax
import jax.numpy as jnp
from jax import lax
from jax.experimental import pallas as pl
from jax.experimental.pallas import tpu as pltpu
from jax.experimental.pallas import tpu_sc as plsc

N = 10000
E = 320000
R = 16
NBASES = 4
D = 128

BN = 1000
NBLK = N // BN

CH = 128
NCHUNK = E // CH
NWORK = 32

NPAD = 10240
ROWS_PER_TILE = NPAD // 16
DUMP = 128


def _hw_body(comp_ref, bases_ref, wself_ref, h_ref, out_ref):
    r = pl.program_id(1)
    rr = jnp.minimum(r, R - 1)
    w = comp_ref[rr, 0] * bases_ref[0]
    for b in range(1, NBASES):
        w = w + comp_ref[rr, b] * bases_ref[b]
    w = jnp.where(r == R, wself_ref[...], w)
    out_ref[...] = jnp.dot(h_ref[...], w, preferred_element_type=jnp.float32)


def _hw_call(comp, bases, wself, h):
    return pl.pallas_call(
        _hw_body,
        grid=(NBLK, R + 1),
        in_specs=[
            pl.BlockSpec(memory_space=pltpu.SMEM),
            pl.BlockSpec((NBASES, D, D), lambda i, r: (0, 0, 0)),
            pl.BlockSpec((D, D), lambda i, r: (0, 0)),
            pl.BlockSpec((BN, D), lambda i, r: (i, 0)),
        ],
        out_specs=pl.BlockSpec((None, BN, D), lambda i, r: (r, i, 0)),
        out_shape=jax.ShapeDtypeStruct((R + 1, N, D), jnp.float32),
    )(comp, bases, wself, h)


def _sc_edges_body(ei, et, hw, out, src_v, et_v, gidx_v, dst_v, rows_v,
                   stage_v, agg_sh, sem):
    c = lax.axis_index("c")
    s = lax.axis_index("s")
    wid = c * 16 + s

    def _zero(i, carry):
        for k in range(D // 16):
            stage_v[i, pl.ds(k * 16, 16)] = jnp.zeros((16,), jnp.float32)
        return carry
    lax.fori_loop(0, DUMP, _zero, 0)
    for j in range(ROWS_PER_TILE // DUMP):
        pltpu.sync_copy(stage_v,
                        agg_sh.at[pl.ds(s * ROWS_PER_TILE + j * DUMP, DUMP)])
    plsc.subcore_barrier()

    start = wid * NCHUNK // NWORK
    stop = (wid + 1) * NCHUNK // NWORK

    def _chunk(ci, carry):
        off = ci * CH
        pltpu.sync_copy(ei.at[0, pl.ds(off, CH)], src_v)
        pltpu.sync_copy(ei.at[1, pl.ds(off, CH)], dst_v)
        pltpu.sync_copy(et.at[pl.ds(off, CH)], et_v)
        for i in range(CH // 16):
            sl = pl.ds(i * 16, 16)
            gidx_v[sl] = et_v[sl] * N + src_v[sl]
        pltpu.async_copy(hw.at[gidx_v], rows_v, sem).wait()
        pltpu.sync_copy(rows_v, agg_sh.at[dst_v], add=True)
        return carry
    lax.fori_loop(start, stop, _chunk, 0)
    plsc.subcore_barrier()

    for j in range(ROWS_PER_TILE // DUMP):
        row0 = s * ROWS_PER_TILE + j * DUMP
        pltpu.sync_copy(agg_sh.at[pl.ds(row0, DUMP)], stage_v)
        pltpu.sync_copy(stage_v, out.at[c, pl.ds(row0, DUMP)])


def _sc_edges_call(edge_index, etypes, hw_flat):
    mesh = plsc.VectorSubcoreMesh(core_axis_name="c", subcore_axis_name="s")
    f = functools.partial(
        pl.kernel,
        out_type=jax.ShapeDtypeStruct((2, NPAD, D), jnp.float32),
        mesh=mesh,
        scratch_types=[
            pltpu.VMEM((CH,), jnp.int32),
            pltpu.VMEM((CH,), jnp.int32),
            pltpu.VMEM((CH,), jnp.int32),
            pltpu.VMEM((CH,), jnp.int32),
            pltpu.VMEM((CH, D), jnp.float32),
            pltpu.VMEM((DUMP, D), jnp.float32),
            pltpu.VMEM_SHARED((NPAD, D), jnp.float32),
            pltpu.SemaphoreType.DMA,
        ],
    )(_sc_edges_body)
    return f(edge_index, etypes, hw_flat)


def _combine_body(agg_ref, self_ref, bias_ref, gamma_ref, beta_ref, out_ref,
                  *, act):
    x = agg_ref[0] + agg_ref[1] + self_ref[...] + bias_ref[...]
    mu = jnp.mean(x, axis=-1, keepdims=True)
    xc = x - mu
    var = jnp.mean(xc * xc, axis=-1, keepdims=True)
    y = gamma_ref[...] * (xc * lax.rsqrt(var + 1e-5)) + beta_ref[...]
    if act:
        y = jnp.maximum(y, 0.0)
    out_ref[...] = y


def _combine_call(agg, selfpart, bias, gamma, beta, act):
    return pl.pallas_call(
        functools.partial(_combine_body, act=act),
        grid=(NBLK,),
        in_specs=[
            pl.BlockSpec((2, BN, D), lambda i: (0, i, 0)),
            pl.BlockSpec((BN, D), lambda i: (i, 0)),
            pl.BlockSpec((1, D), lambda i: (0, 0)),
            pl.BlockSpec((1, D), lambda i: (0, 0)),
            pl.BlockSpec((1, D), lambda i: (0, 0)),
        ],
        out_specs=pl.BlockSpec((BN, D), lambda i: (i, 0)),
        out_shape=jax.ShapeDtypeStruct((N, D), jnp.float32),
    )(agg, selfpart, bias.reshape(1, D), gamma.reshape(1, D),
      beta.reshape(1, D))


GB = 320
GC = 64


def _sc_gather_body(nodes, h2, out, idx_v, rows_v, sem):
    c = lax.axis_index("c")
    s = lax.axis_index("s")
    wid = c * 16 + s
    base = jnp.minimum(wid * GB, N - GB)
    for j in range(GB // GC):
        pltpu.sync_copy(nodes.at[pl.ds(base + j * GC, GC)], idx_v)
        pltpu.async_copy(h2.at[idx_v], rows_v, sem).wait()
        pltpu.sync_copy(rows_v, out.at[pl.ds(base + j * GC, GC)])


def _sc_gather_call(nodes, h2):
    mesh = plsc.VectorSubcoreMesh(core_axis_name="c", subcore_axis_name="s")
    f = functools.partial(
        pl.kernel,
        out_type=jax.ShapeDtypeStruct((N, D), jnp.float32),
        mesh=mesh,
        scratch_types=[
            pltpu.VMEM((GC,), jnp.int32),
            pltpu.VMEM((GC, D), jnp.float32),
            pltpu.SemaphoreType.DMA,
        ],
    )(_sc_gather_body)
    return f(nodes, h2)


def _layer(h, edge_index, etypes, bases, comp, wself, bias, gamma, beta, act):
    hw = _hw_call(comp, bases, wself, h)
    agg = _sc_edges_call(edge_index, etypes, hw.reshape((R + 1) * N, D))
    return _combine_call(agg[:, :N], hw[R], bias, gamma, beta, act)


def kernel(nodes, edge_index, etypes, node_feat, bases0, comp0, wself0,
           bias0, gamma0, beta0, bases1, comp1, wself1, bias1, gamma1,
           beta1):
    h1 = _layer(node_feat, edge_index, etypes, bases0, comp0, wself0,
                bias0, gamma0, beta0, True)
    h2 = _layer(h1, edge_index, etypes, bases1, comp1, wself1,
                bias1, gamma1, beta1, False)
    return _sc_gather_call(nodes, h2)
